# baseline (speedup 1.0000x reference)
import functools

import jax
import jax.numpy as jnp
import numpy as np
from jax.experimental import pallas as pl
from jax.experimental.pallas import tpu as pltpu
from jax.sharding import Mesh, PartitionSpec as P

shard_map = jax.shard_map

_DP = 384


def _body(off_ref, f_ref, sp_ref, dt_ref, cnt_ref, tot_ref, u_ref):
    k = pl.program_id(0)
    nb, n, d = f_ref.shape

    @pl.when(k == 0)
    def _prep():
        cnt_ref[0, 0] = 0.0
        tot_ref[0, 0] = 0.0
        lane = jax.lax.broadcasted_iota(jnp.int32, (n, _DP), 1)
        aug = jnp.where(lane == d, 1.0, 0.0)
        for b in range(nb):
            f = f_ref[b]
            ssq = jnp.sum(f * f, axis=-1, keepdims=True)
            inv = jax.lax.rsqrt(jnp.maximum(ssq, 1e-24))
            padded = jnp.pad(f * inv, ((0, 0), (0, _DP - d)))
            u_ref[b] = (padded + aug).astype(u_ref.dtype)

    @pl.when(k > 0)
    def _pairs():
        sp = sp_ref[0]
        xs = sp[:, 0:1]
        ys = sp[:, 1:2]
        a2 = xs * xs + ys * ys

        ub = u_ref[off_ref[0, 0] + k - 1]
        dims = (((1,), (0,)), ((), ()))
        cnt = 0.0
        mcos = 0.0
        for p in range(nb):
            dt = dt_ref[p]
            xd = dt[0:1, :]
            yd = dt[1:2, :]
            xd2 = xd + xd
            yd2 = yd + yd
            rb = xd * xd + yd * yd - 1.0
            m = ((a2 + rb) <= (xs * xd2 + ys * yd2)).astype(jnp.bfloat16)
            v = jax.lax.dot_general(m, ub, dims,
                                    preferred_element_type=jnp.float32)
            cnt += jnp.sum(v[:, d:d + 1])
            mcos += jnp.sum(v[:, :d] * u_ref[p, :, :d].astype(jnp.float32))
        cnt_ref[0, 0] += cnt
        tot_ref[0, 0] += cnt - mcos


def _run(features, src_p, dst_t, off, b_local):
    B, N, D = features.shape
    return pl.pallas_call(
        _body,
        grid=(b_local + 1,),
        in_specs=[
            pl.BlockSpec(memory_space=pltpu.SMEM),
            pl.BlockSpec((B, N, D), lambda k: (0, 0, 0)),
            pl.BlockSpec((1, N, 8),
                         lambda k: (jnp.maximum(k - 1, 0), 0, 0)),
            pl.BlockSpec((B, 8, N),
                         lambda k: (jnp.maximum(k - 1, 0), 0, 0)),
        ],
        out_specs=[
            pl.BlockSpec(memory_space=pltpu.SMEM),
            pl.BlockSpec(memory_space=pltpu.SMEM),
        ],
        out_shape=[
            jax.ShapeDtypeStruct((1, 1), jnp.float32),
            jax.ShapeDtypeStruct((1, 1), jnp.float32),
        ],
        scratch_shapes=[pltpu.VMEM((B, N, _DP), jnp.bfloat16)],
        compiler_params=pltpu.CompilerParams(
            dimension_semantics=("arbitrary",)),
    )(off, features, src_p, dst_t)


def kernel(features, pts_src, pts_dst, invis_idx, height, width):
    del invis_idx
    B, N, D = features.shape
    radius = 1.0
    fx = (jnp.asarray(width, jnp.float32) - 1.0) / 2.0
    fy = (jnp.asarray(height, jnp.float32) - 1.0) / 2.0
    factor = jnp.stack([fx, fy]) / radius

    src_p = jnp.pad((pts_src + 1.0) * factor,
                    ((0, 0), (0, 0), (0, 6)))
    dst_t = jnp.pad(jnp.transpose(
        (pts_dst.reshape(B * B, N, 2) + 1.0) * factor, (0, 2, 1)),
        ((0, 0), (0, 6), (0, 0)))

    devs = jax.devices()
    nd = 2 if len(devs) >= 2 and B % 2 == 0 else 1
    if nd == 1:
        off = jnp.zeros((1, 1), jnp.int32)
        cnt, tot = _run(features, src_p, dst_t, off, B)
        return tot[0, 0] / cnt[0, 0]

    b_local = B // nd
    mesh = Mesh(np.array(devs[:nd]), ("x",))

    def shard_fn(feats, sp, dt):
        off = jnp.reshape(
            jax.lax.axis_index("x") * b_local, (1, 1)).astype(jnp.int32)
        return _run(feats, sp, dt, off, b_local)

    cnt, tot = shard_map(
        shard_fn, mesh=mesh,
        in_specs=(P(), P("x"), P("x")),
        out_specs=(P("x"), P("x")),
        check_vma=False)(features, src_p, dst_t)
    return jnp.sum(tot) / jnp.sum(cnt)


# speedup vs baseline: 15.2126x; 15.2126x over previous
import functools

import jax
import jax.numpy as jnp
from jax.experimental import pallas as pl
from jax.experimental.pallas import tpu as pltpu

_DP = 384


def _body(f_ref, sp_ref, dt_ref, cnt_ref, tot_ref, u_ref, u32_ref):
    k = pl.program_id(0)
    nb, n, d = f_ref.shape

    @pl.when(k == 0)
    def _prep():
        cnt_ref[0, 0] = 0.0
        tot_ref[0, 0] = 0.0
        lane = jax.lax.broadcasted_iota(jnp.int32, (n, _DP), 1)
        aug = jnp.where(lane == d, 1.0, 0.0)
        for b in range(nb):
            f = f_ref[b]
            ssq = jnp.sum(f * f, axis=-1, keepdims=True)
            inv = jax.lax.rsqrt(jnp.maximum(ssq, 1e-24))
            un = f * inv
            u32_ref[b] = un
            padded = jnp.pad(un, ((0, 0), (0, _DP - d)))
            u_ref[b] = (padded + aug).astype(u_ref.dtype)

    @pl.when(k > 0)
    def _pairs():
        sp = sp_ref[0]
        xs = sp[:, 0:1]
        ys = sp[:, 1:2]
        a2 = xs * xs + ys * ys

        ub = u_ref[k - 1]
        dims = (((1,), (0,)), ((), ()))
        ms = []
        for p in range(nb):
            dt = dt_ref[p]
            xd = dt[0:1, :]
            yd = dt[1:2, :]
            xd2 = xd + xd
            yd2 = yd + yd
            rb = xd * xd + yd * yd - 1.0
            ms.append(((a2 + rb) <= (xs * xd2 + ys * yd2))
                      .astype(jnp.bfloat16))
        mall = jnp.concatenate(ms, axis=0)
        v = jax.lax.dot_general(mall, ub, dims,
                                preferred_element_type=jnp.float32)
        cnt = jnp.sum(v[:, d:d + 1])
        ua = u32_ref[:, :, :].reshape(nb * n, d)
        mcos = jnp.sum(v[:, :d] * ua)
        cnt_ref[0, 0] += cnt
        tot_ref[0, 0] += cnt - mcos


def kernel(features, pts_src, pts_dst, invis_idx, height, width):
    del invis_idx
    B, N, D = features.shape
    radius = 1.0
    fx = (jnp.asarray(width, jnp.float32) - 1.0) / 2.0
    fy = (jnp.asarray(height, jnp.float32) - 1.0) / 2.0
    factor = jnp.stack([fx, fy]) / radius

    src_p = jnp.pad((pts_src + 1.0) * factor,
                    ((0, 0), (0, 0), (0, 6)))
    dst_t = jnp.pad(jnp.transpose(
        (pts_dst.reshape(B * B, N, 2) + 1.0) * factor, (0, 2, 1)),
        ((0, 0), (0, 6), (0, 0)))

    cnt, tot = pl.pallas_call(
        _body,
        grid=(B + 1,),
        in_specs=[
            pl.BlockSpec((B, N, D), lambda k: (0, 0, 0)),
            pl.BlockSpec((1, N, 8),
                         lambda k: (jnp.maximum(k - 1, 0), 0, 0)),
            pl.BlockSpec((B, 8, N),
                         lambda k: (jnp.maximum(k - 1, 0), 0, 0)),
        ],
        out_specs=[
            pl.BlockSpec(memory_space=pltpu.SMEM),
            pl.BlockSpec(memory_space=pltpu.SMEM),
        ],
        out_shape=[
            jax.ShapeDtypeStruct((1, 1), jnp.float32),
            jax.ShapeDtypeStruct((1, 1), jnp.float32),
        ],
        scratch_shapes=[pltpu.VMEM((B, N, _DP), jnp.bfloat16),
                        pltpu.VMEM((B, N, D), jnp.float32)],
        compiler_params=pltpu.CompilerParams(
            dimension_semantics=("arbitrary",)),
    )(features, src_p, dst_t)

    return tot[0, 0] / cnt[0, 0]


# speedup vs baseline: 15.4840x; 1.0178x over previous
import functools

import jax
import jax.numpy as jnp
from jax.experimental import pallas as pl
from jax.experimental.pallas import tpu as pltpu

_DP = 384


def _body(f_ref, sp_ref, dt_ref, cnt_ref, tot_ref, u_ref, u32_ref):
    k = pl.program_id(0)
    nb, n, d = f_ref.shape

    @pl.when(k == 0)
    def _prep():
        cnt_ref[0, 0] = 0.0
        tot_ref[0, 0] = 0.0
        lane = jax.lax.broadcasted_iota(jnp.int32, (n, _DP), 1)
        aug = jnp.where(lane == d, 1.0, 0.0)
        for b in range(nb):
            f = f_ref[b]
            ssq = jnp.sum(f * f, axis=-1, keepdims=True)
            inv = jax.lax.rsqrt(jnp.maximum(ssq, 1e-24))
            un = f * inv
            u32_ref[b] = un
            padded = jnp.pad(un, ((0, 0), (0, _DP - d)))
            u_ref[b] = (padded + aug).astype(u_ref.dtype)

    @pl.when(k > 0)
    def _pairs():
        sp = sp_ref[0]
        xs = sp[:, 0:1]
        ys = sp[:, 1:2]
        a2 = xs * xs + ys * ys

        ub = u_ref[k - 1]
        dims = (((1,), (0,)), ((), ()))
        cnt = 0.0
        mcos = 0.0
        for p in range(nb):
            dt = dt_ref[p]
            xd = dt[0:1, :]
            yd = dt[1:2, :]
            xd2 = xd + xd
            yd2 = yd + yd
            rb = xd * xd + yd * yd - 1.0
            m = ((a2 + rb) <= (xs * xd2 + ys * yd2)).astype(jnp.bfloat16)
            v = jax.lax.dot_general(m, ub, dims,
                                    preferred_element_type=jnp.float32)
            cnt += jnp.sum(v[:, d:d + 1])
            mcos += jnp.sum(v[:, :d] * u32_ref[p])
        cnt_ref[0, 0] += cnt
        tot_ref[0, 0] += cnt - mcos


def kernel(features, pts_src, pts_dst, invis_idx, height, width):
    del invis_idx
    B, N, D = features.shape
    radius = 1.0
    fx = (jnp.asarray(width, jnp.float32) - 1.0) / 2.0
    fy = (jnp.asarray(height, jnp.float32) - 1.0) / 2.0
    factor = jnp.stack([fx, fy]) / radius

    src_p = jnp.pad((pts_src + 1.0) * factor,
                    ((0, 0), (0, 0), (0, 6)))
    dst_t = jnp.pad(jnp.transpose(
        (pts_dst.reshape(B * B, N, 2) + 1.0) * factor, (0, 2, 1)),
        ((0, 0), (0, 6), (0, 0)))

    cnt, tot = pl.pallas_call(
        _body,
        grid=(B + 1,),
        in_specs=[
            pl.BlockSpec((B, N, D), lambda k: (0, 0, 0)),
            pl.BlockSpec((1, N, 8),
                         lambda k: (jnp.maximum(k - 1, 0), 0, 0)),
            pl.BlockSpec((B, 8, N),
                         lambda k: (jnp.maximum(k - 1, 0), 0, 0)),
        ],
        out_specs=[
            pl.BlockSpec(memory_space=pltpu.SMEM),
            pl.BlockSpec(memory_space=pltpu.SMEM),
        ],
        out_shape=[
            jax.ShapeDtypeStruct((1, 1), jnp.float32),
            jax.ShapeDtypeStruct((1, 1), jnp.float32),
        ],
        scratch_shapes=[pltpu.VMEM((B, N, _DP), jnp.bfloat16),
                        pltpu.VMEM((B, N, D), jnp.float32)],
        compiler_params=pltpu.CompilerParams(
            dimension_semantics=("arbitrary",)),
    )(features, src_p, dst_t)

    return tot[0, 0] / cnt[0, 0]


# speedup vs baseline: 15.9010x; 1.0269x over previous
import functools

import jax
import jax.numpy as jnp
from jax.experimental import pallas as pl
from jax.experimental.pallas import tpu as pltpu

_DP = 384


def _body(f_ref, sp_ref, dt_ref, cnt_ref, tot_ref, u_ref, u32_ref):
    k = pl.program_id(0)
    nb, n, d = f_ref.shape

    @pl.when(k == 0)
    def _prep():
        cnt_ref[0, 0] = 0.0
        tot_ref[0, 0] = 0.0
        lane = jax.lax.broadcasted_iota(jnp.int32, (n, _DP), 1)
        aug = jnp.where(lane == d, 1.0, 0.0)
        for b in range(nb):
            f = f_ref[b]
            ssq = jnp.sum(f * f, axis=-1, keepdims=True)
            inv = jax.lax.rsqrt(jnp.maximum(ssq, 1e-24))
            un = f * inv
            u32_ref[b] = un
            padded = jnp.pad(un, ((0, 0), (0, _DP - d)))
            u_ref[b] = (padded + aug).astype(u_ref.dtype)

    @pl.when(k > 0)
    def _pairs():
        sp = sp_ref[0]
        xs = sp[:, 0:1]
        ys = sp[:, 1:2]
        a2 = xs * xs + ys * ys

        ub = u_ref[k - 1]
        dims = (((1,), (0,)), ((), ()))
        cnt = 0.0
        mcos = 0.0
        for p in range(nb):
            dt = dt_ref[p]
            xd = dt[0:1, :]
            yd = dt[1:2, :]
            xd2 = xd + xd
            yd2 = yd + yd
            rb = xd * xd + yd * yd - 1.0
            m = ((a2 + rb) <= (xs * xd2 + ys * yd2)).astype(jnp.float8_e4m3fn)
            v = jax.lax.dot_general(m, ub, dims,
                                    preferred_element_type=jnp.float32)
            cnt += jnp.sum(v[:, d:d + 1])
            mcos += jnp.sum(v[:, :d] * u32_ref[p])
        cnt_ref[0, 0] += cnt
        tot_ref[0, 0] += cnt - mcos


def kernel(features, pts_src, pts_dst, invis_idx, height, width):
    del invis_idx
    B, N, D = features.shape
    radius = 1.0
    fx = (jnp.asarray(width, jnp.float32) - 1.0) / 2.0
    fy = (jnp.asarray(height, jnp.float32) - 1.0) / 2.0
    factor = jnp.stack([fx, fy]) / radius

    src_p = jnp.pad((pts_src + 1.0) * factor,
                    ((0, 0), (0, 0), (0, 6)))
    dst_t = jnp.pad(jnp.transpose(
        (pts_dst.reshape(B * B, N, 2) + 1.0) * factor, (0, 2, 1)),
        ((0, 0), (0, 6), (0, 0)))

    cnt, tot = pl.pallas_call(
        _body,
        grid=(B + 1,),
        in_specs=[
            pl.BlockSpec((B, N, D), lambda k: (0, 0, 0)),
            pl.BlockSpec((1, N, 8),
                         lambda k: (jnp.maximum(k - 1, 0), 0, 0)),
            pl.BlockSpec((B, 8, N),
                         lambda k: (jnp.maximum(k - 1, 0), 0, 0)),
        ],
        out_specs=[
            pl.BlockSpec(memory_space=pltpu.SMEM),
            pl.BlockSpec(memory_space=pltpu.SMEM),
        ],
        out_shape=[
            jax.ShapeDtypeStruct((1, 1), jnp.float32),
            jax.ShapeDtypeStruct((1, 1), jnp.float32),
        ],
        scratch_shapes=[pltpu.VMEM((B, N, _DP), jnp.float8_e4m3fn),
                        pltpu.VMEM((B, N, D), jnp.float32)],
        compiler_params=pltpu.CompilerParams(
            dimension_semantics=("arbitrary",)),
    )(features, src_p, dst_t)

    return tot[0, 0] / cnt[0, 0]
